# baseline (speedup 1.0000x reference)
import math

import jax
import jax.numpy as jnp
from jax.experimental import pallas as pl
from jax.experimental.pallas import tpu as pltpu

D_MODEL = 256
GRID_DIM = 30
G = 9
R = 3
NBUF = 4
NROWS = 32 * G
NCHUNK = NROWS // R


def _body(x_ref, io_ref, pair_ref, o_ref, in_bufs, out_bufs, comb, in_sem, out_sem):
    def in_copy(i, slot):
        return pltpu.make_async_copy(
            x_ref.at[pl.ds(i * R, R)], in_bufs.at[slot], in_sem.at[slot])

    def out_copy(i, slot):
        return pltpu.make_async_copy(
            out_bufs.at[slot], o_ref.at[pl.ds(i * R, R)], out_sem.at[slot])

    for k in range(NBUF):
        in_copy(k, k).start()

    dim = D_MODEL // 2
    h = jax.lax.broadcasted_iota(jnp.int32, (GRID_DIM, GRID_DIM, 2 * dim), 0)
    w = jax.lax.broadcasted_iota(jnp.int32, (GRID_DIM, GRID_DIM, 2 * dim), 1)
    c = jax.lax.broadcasted_iota(jnp.int32, (GRID_DIM, GRID_DIM, 2 * dim), 2)
    pos = jnp.where(c < dim, h, w).astype(jnp.float32)
    cl = c % dim
    freq = jnp.exp((cl - cl % 2).astype(jnp.float32) * (-math.log(10000.0) / dim))
    angle = pos * freq
    pos_emb = jnp.where(cl % 2 == 0, jnp.sin(angle), jnp.cos(angle))
    for gg in range(G):
        comb[gg, :, :, 0:256] = pos_emb
        comb[gg, :, :, 256:320] = jnp.broadcast_to(
            io_ref[gg % 2, :][None, None, :], (GRID_DIM, GRID_DIM, 64))
        comb[gg, :, :, 320:384] = jnp.broadcast_to(
            pair_ref[gg // 2, :][None, None, :], (GRID_DIM, GRID_DIM, 64))

    def step(i, _):
        slot = jax.lax.rem(i, NBUF)
        in_copy(i, slot).wait()

        @pl.when(i >= NBUF)
        def _():
            out_copy(i - NBUF, slot).wait()

        base = jax.lax.rem(R * i, G)
        out_bufs[slot] = in_bufs[slot] + comb[pl.ds(base, R)]

        @pl.when(i + NBUF < NCHUNK)
        def _():
            in_copy(i + NBUF, slot).start()

        out_copy(i, slot).start()
        return 0

    jax.lax.fori_loop(0, NCHUNK, step, 0)

    for i in range(NCHUNK - NBUF, NCHUNK):
        out_copy(i, i % NBUF).wait()


@jax.jit
def kernel(x, io_table, pair_table):
    B, Gd, H, W, C = x.shape
    xf = x.reshape(B * Gd, H, W, C)
    out = pl.pallas_call(
        _body,
        in_specs=[
            pl.BlockSpec(memory_space=pltpu.HBM),
            pl.BlockSpec(memory_space=pltpu.VMEM),
            pl.BlockSpec(memory_space=pltpu.VMEM),
        ],
        out_specs=pl.BlockSpec(memory_space=pltpu.HBM),
        out_shape=jax.ShapeDtypeStruct((B * Gd, H, W, C), x.dtype),
        scratch_shapes=[
            pltpu.VMEM((NBUF, R, H, W, C), jnp.float32),
            pltpu.VMEM((NBUF, R, H, W, C), jnp.float32),
            pltpu.VMEM((G, H, W, C), jnp.float32),
            pltpu.SemaphoreType.DMA((NBUF,)),
            pltpu.SemaphoreType.DMA((NBUF,)),
        ],
    )(xf, io_table, pair_table)
    return out.reshape(B, Gd, H, W, C)


# speedup vs baseline: 1.0156x; 1.0156x over previous
import math

import jax
import jax.numpy as jnp
from jax.experimental import pallas as pl
from jax.experimental.pallas import tpu as pltpu
from jax.experimental.pallas import tpu_sc as plsc

D_MODEL = 256
GRID_DIM = 30
G = 9
GPB = 3


def _sc_gather_body(io_ref, pair_ref, ge_ref, buf):
    c = jax.lax.axis_index("c")
    s = jax.lax.axis_index("s")

    @pl.when(jnp.logical_and(c == 0, s == 0))
    def _():
        q = D_MODEL // 4
        for gg in range(G):
            pltpu.sync_copy(io_ref.at[pl.ds((gg % 2) * q, q)], buf)
            pltpu.sync_copy(buf, ge_ref.at[pl.ds(gg * 2 * q, q)])
            pltpu.sync_copy(pair_ref.at[pl.ds((gg // 2) * q, q)], buf)
            pltpu.sync_copy(buf, ge_ref.at[pl.ds(gg * 2 * q + q, q)])


def _tc_body(x_ref, ge_ref, o_ref, comb):
    b = pl.program_id(0)
    j = pl.program_id(1)

    @pl.when(jnp.logical_and(b == 0, j == 0))
    def _init():
        dim = D_MODEL // 2
        h = jax.lax.broadcasted_iota(jnp.int32, (GRID_DIM, GRID_DIM, 2 * dim), 0)
        w = jax.lax.broadcasted_iota(jnp.int32, (GRID_DIM, GRID_DIM, 2 * dim), 1)
        c = jax.lax.broadcasted_iota(jnp.int32, (GRID_DIM, GRID_DIM, 2 * dim), 2)
        pos = jnp.where(c < dim, h, w).astype(jnp.float32)
        cl = c % dim
        freq = jnp.exp((cl - cl % 2).astype(jnp.float32) * (-math.log(10000.0) / dim))
        angle = pos * freq
        pos_emb = jnp.where(cl % 2 == 0, jnp.sin(angle), jnp.cos(angle))
        for gg in range(G):
            comb[gg, :, :, 0:256] = pos_emb
            comb[gg, :, :, 256:384] = jnp.broadcast_to(
                ge_ref[gg, :][None, None, :], (GRID_DIM, GRID_DIM, 2 * dim // 2))

    o_ref[0] = x_ref[0] + comb[pl.ds(GPB * j, GPB), :, :, :]


@jax.jit
def kernel(x, io_table, pair_table):
    B, Gd, H, W, C = x.shape
    ge = pl.kernel(
        _sc_gather_body,
        out_type=jax.ShapeDtypeStruct((G * 2 * (D_MODEL // 4),), jnp.float32),
        mesh=plsc.VectorSubcoreMesh(core_axis_name="c", subcore_axis_name="s"),
        scratch_types=[pltpu.VMEM((D_MODEL // 4,), jnp.float32)],
    )(io_table.reshape(-1), pair_table.reshape(-1))
    ge = ge.reshape(G, 2 * (D_MODEL // 4))
    return pl.pallas_call(
        _tc_body,
        grid=(B, Gd // GPB),
        in_specs=[
            pl.BlockSpec((1, GPB, H, W, C), lambda b, j: (b, j, 0, 0, 0)),
            pl.BlockSpec(memory_space=pltpu.VMEM),
        ],
        out_specs=pl.BlockSpec((1, GPB, H, W, C), lambda b, j: (b, j, 0, 0, 0)),
        out_shape=jax.ShapeDtypeStruct((B, Gd, H, W, C), x.dtype),
        scratch_shapes=[
            pltpu.VMEM((G, H, W, C), jnp.float32),
        ],
    )(x, ge)


# speedup vs baseline: 1.0302x; 1.0144x over previous
import math

import jax
import jax.numpy as jnp
from jax.experimental import pallas as pl
from jax.experimental.pallas import tpu as pltpu
from jax.experimental.pallas import tpu_sc as plsc

D_MODEL = 256
GRID_DIM = 30
G = 9
GPB = 3


def _sc_gather_body(io_ref, pair_ref, ge_ref, buf):
    c = jax.lax.axis_index("c")
    s = jax.lax.axis_index("s")

    @pl.when(jnp.logical_and(c == 0, s == 0))
    def _():
        q = D_MODEL // 4
        for gg in range(G):
            pltpu.sync_copy(io_ref.at[pl.ds((gg % 2) * q, q)], buf)
            pltpu.sync_copy(buf, ge_ref.at[pl.ds(gg * 2 * q, q)])
            pltpu.sync_copy(pair_ref.at[pl.ds((gg // 2) * q, q)], buf)
            pltpu.sync_copy(buf, ge_ref.at[pl.ds(gg * 2 * q + q, q)])


def _tc_body(x_ref, ge_ref, o_ref, comb):
    b = pl.program_id(0)

    @pl.when(b == 0)
    def _init():
        dim = D_MODEL // 2
        h = jax.lax.broadcasted_iota(jnp.int32, (GRID_DIM, GRID_DIM, 2 * dim), 0)
        w = jax.lax.broadcasted_iota(jnp.int32, (GRID_DIM, GRID_DIM, 2 * dim), 1)
        c = jax.lax.broadcasted_iota(jnp.int32, (GRID_DIM, GRID_DIM, 2 * dim), 2)
        pos = jnp.where(c < dim, h, w).astype(jnp.float32)
        cl = c % dim
        freq = jnp.exp((cl - cl % 2).astype(jnp.float32) * (-math.log(10000.0) / dim))
        angle = pos * freq
        pos_emb = jnp.where(cl % 2 == 0, jnp.sin(angle), jnp.cos(angle))
        for gg in range(G):
            comb[gg, :, :, 0:256] = pos_emb
            comb[gg, :, :, 256:384] = jnp.broadcast_to(
                ge_ref[gg, :][None, None, :], (GRID_DIM, GRID_DIM, 2 * dim // 2))

    o_ref[0] = x_ref[0] + comb[...]


@jax.jit
def kernel(x, io_table, pair_table):
    B, Gd, H, W, C = x.shape
    ge = pl.kernel(
        _sc_gather_body,
        out_type=jax.ShapeDtypeStruct((G * 2 * (D_MODEL // 4),), jnp.float32),
        mesh=plsc.VectorSubcoreMesh(core_axis_name="c", subcore_axis_name="s"),
        scratch_types=[pltpu.VMEM((D_MODEL // 4,), jnp.float32)],
    )(io_table.reshape(-1), pair_table.reshape(-1))
    ge = ge.reshape(G, 2 * (D_MODEL // 4))
    return pl.pallas_call(
        _tc_body,
        grid=(B,),
        in_specs=[
            pl.BlockSpec((1, Gd, H, W, C), lambda b: (b, 0, 0, 0, 0)),
            pl.BlockSpec(memory_space=pltpu.VMEM),
        ],
        out_specs=pl.BlockSpec((1, Gd, H, W, C), lambda b: (b, 0, 0, 0, 0)),
        out_shape=jax.ShapeDtypeStruct((B, Gd, H, W, C), x.dtype),
        scratch_shapes=[
            pltpu.VMEM((G, H, W, C), jnp.float32),
        ],
        compiler_params=pltpu.CompilerParams(
            vmem_limit_bytes=100 * 1024 * 1024),
    )(x, ge)
